# baseline (speedup 1.0000x reference)
import functools

import jax
import jax.numpy as jnp
from jax import lax
from jax.experimental import pallas as pl
from jax.experimental.pallas import tpu as pltpu
from jax.experimental.pallas import tpu_sc as plsc

K_CODES = 8192
D_CH = 256
N_BATCH = 8
N_HW = 1024
KT = 512
COMMIT = 0.25


def _argmin_body(x_ref, w_ref, idx_ref, loss_ref, s2_ref, en8_ref):
    b = pl.program_id(0)

    @pl.when(b == 0)
    def _():
        loss_ref[0, 0] = 0.0
        for kt in range(K_CODES // KT):
            wt = w_ref[kt * KT:(kt + 1) * KT, :]
            en = jnp.sum(wt * wt, axis=1, keepdims=True)
            en8_ref[kt * (KT // 8):(kt + 1) * (KT // 8)] = en.reshape(KT // 8, 8, 1)

    xb = x_ref[0]
    xn = jnp.sum(xb * xb, axis=0, keepdims=True)
    xm2 = -2.0 * xb

    m8 = jnp.full((8, N_HW), jnp.float32(jnp.inf))
    r8 = jnp.zeros((8, N_HW), jnp.int32)
    vc_per_kt = KT // 8

    for kt in range(K_CODES // KT):
        wt = w_ref[kt * KT:(kt + 1) * KT, :]
        buf = kt % 2
        s2_ref[buf] = jnp.dot(wt, xm2, preferred_element_type=jnp.float32)

        def chunk(v, carry):
            m8c, r8c = carry
            s2v = s2_ref[buf, pl.ds(v * 8, 8), :]
            env = en8_ref[kt * vc_per_kt + v]
            d = (xn + env) + s2v
            take = d < m8c
            m8c = jnp.where(take, d, m8c)
            r8c = jnp.where(take, kt * vc_per_kt + v, r8c)
            return m8c, r8c

        m8, r8 = lax.fori_loop(0, vc_per_kt, chunk, (m8, r8), unroll=8)

    tmin = jnp.min(m8, axis=0, keepdims=True)
    siota = lax.broadcasted_iota(jnp.int32, (8, N_HW), 0)
    gk = r8 * 8 + siota
    cand = jnp.where(m8 == tmin, gk, jnp.int32(2 ** 30))
    gidx = jnp.min(cand, axis=0, keepdims=True)
    idx_ref[0] = gidx
    loss_ref[0, 0] += jnp.sum(tmin)


def _distance_argmin(x3, w):
    return pl.pallas_call(
        _argmin_body,
        grid=(N_BATCH,),
        in_specs=[
            pl.BlockSpec((1, D_CH, N_HW), lambda b: (b, 0, 0)),
            pl.BlockSpec((K_CODES, D_CH), lambda b: (0, 0)),
        ],
        out_specs=[
            pl.BlockSpec((1, 1, N_HW), lambda b: (b, 0, 0)),
            pl.BlockSpec(memory_space=pltpu.SMEM),
        ],
        out_shape=[
            jax.ShapeDtypeStruct((N_BATCH, 1, N_HW), jnp.int32),
            jax.ShapeDtypeStruct((1, 1), jnp.float32),
        ],
        scratch_shapes=[
            pltpu.VMEM((2, KT, N_HW), jnp.float32),
            pltpu.VMEM((K_CODES // 8, 8, 1), jnp.float32),
        ],
    )(x3, w)


@functools.cache
def _make_sc_gather():
    info = plsc.get_sparse_core_info()
    nw = info.num_cores * info.num_subcores
    b_per_w = (N_BATCH * N_HW) // nw
    mesh = plsc.VectorSubcoreMesh(core_axis_name="c", subcore_axis_name="s")

    @functools.partial(
        pl.kernel, mesh=mesh,
        out_type=jax.ShapeDtypeStruct((N_BATCH * N_HW, D_CH), jnp.float32),
        scratch_types=[
            pltpu.VMEM((b_per_w,), jnp.int32),
            pltpu.VMEM((b_per_w, D_CH), jnp.float32),
            pltpu.SemaphoreType.DMA,
        ],
    )
    def sc_gather(table_hbm, idx_hbm, out_hbm, idx_v, rows_v, sem):
        wid = lax.axis_index("s") * info.num_cores + lax.axis_index("c")
        base = wid * b_per_w
        pltpu.sync_copy(idx_hbm.at[pl.ds(base, b_per_w)], idx_v)
        pltpu.async_copy(table_hbm.at[idx_v], rows_v, sem).wait()
        pltpu.sync_copy(rows_v, out_hbm.at[pl.ds(base, b_per_w)])

    return sc_gather


def _transpose_body(q_ref, x_ref, o_ref):
    qt = q_ref[0].T
    xb = x_ref[0]
    o_ref[0] = xb + (qt - xb)


def _transpose_back(q, x3):
    return pl.pallas_call(
        _transpose_body,
        grid=(N_BATCH,),
        in_specs=[
            pl.BlockSpec((1, N_HW, D_CH), lambda b: (b, 0, 0)),
            pl.BlockSpec((1, D_CH, N_HW), lambda b: (b, 0, 0)),
        ],
        out_specs=pl.BlockSpec((1, D_CH, N_HW), lambda b: (b, 0, 0)),
        out_shape=jax.ShapeDtypeStruct((N_BATCH, D_CH, N_HW), jnp.float32),
    )(q, x3)


def kernel(x, embedding_weight):
    x3 = x.reshape(N_BATCH, D_CH, N_HW)
    idx3, loss_sum = _distance_argmin(x3, embedding_weight)
    idx_flat = idx3.reshape(N_BATCH * N_HW)
    q = _make_sc_gather()(embedding_weight, idx_flat)
    out = _transpose_back(q.reshape(N_BATCH, N_HW, D_CH), x3)
    out = out.reshape(N_BATCH, D_CH, 32, 32)
    loss = loss_sum[0, 0] * ((1.0 + COMMIT) / (N_BATCH * N_HW * D_CH))
    return (out, loss, idx_flat[:, None])


# speedup vs baseline: 2.0655x; 2.0655x over previous
import functools

import jax
import jax.numpy as jnp
from jax import lax
from jax.experimental import pallas as pl
from jax.experimental.pallas import tpu as pltpu
from jax.experimental.pallas import tpu_sc as plsc

K_CODES = 8192
D_CH = 256
N_BATCH = 8
N_HW = 1024
KT = 512
COMMIT = 0.25


def _argmin_body(x_ref, w_ref, idx_ref, loss_ref, s2_ref, en8_ref):
    b = pl.program_id(0)

    @pl.when(b == 0)
    def _():
        loss_ref[0, 0] = 0.0
        for kt in range(K_CODES // KT):
            wt = w_ref[kt * KT:(kt + 1) * KT, :]
            en = jnp.sum(wt * wt, axis=1, keepdims=True)
            en8_ref[kt * (KT // 8):(kt + 1) * (KT // 8)] = en.reshape(KT // 8, 8, 1)

    xb = x_ref[0]
    xn = jnp.sum(xb * xb, axis=0, keepdims=True)
    xm2 = -2.0 * xb

    m8 = jnp.full((8, N_HW), jnp.float32(jnp.inf))
    r8 = jnp.zeros((8, N_HW), jnp.int32)
    vc_per_kt = KT // 8

    for kt in range(K_CODES // KT):
        wt = w_ref[kt * KT:(kt + 1) * KT, :]
        buf = kt % 2
        s2_ref[buf] = jnp.dot(wt, xm2, preferred_element_type=jnp.float32)

        for v in range(vc_per_kt):
            s2v = s2_ref[buf, v * 8:(v + 1) * 8, :]
            env = en8_ref[kt * vc_per_kt + v]
            d = (xn + env) + s2v
            take = d < m8
            m8 = jnp.where(take, d, m8)
            r8 = jnp.where(take, jnp.int32(kt * vc_per_kt + v), r8)

    tmin = jnp.min(m8, axis=0, keepdims=True)
    siota = lax.broadcasted_iota(jnp.int32, (8, N_HW), 0)
    gk = r8 * 8 + siota
    cand = jnp.where(m8 == tmin, gk, jnp.int32(2 ** 30))
    gidx = jnp.min(cand, axis=0, keepdims=True)
    idx_ref[0] = gidx
    loss_ref[0, 0] += jnp.sum(tmin)


def _distance_argmin(x3, w):
    return pl.pallas_call(
        _argmin_body,
        grid=(N_BATCH,),
        in_specs=[
            pl.BlockSpec((1, D_CH, N_HW), lambda b: (b, 0, 0)),
            pl.BlockSpec((K_CODES, D_CH), lambda b: (0, 0)),
        ],
        out_specs=[
            pl.BlockSpec((1, 1, N_HW), lambda b: (b, 0, 0)),
            pl.BlockSpec(memory_space=pltpu.SMEM),
        ],
        out_shape=[
            jax.ShapeDtypeStruct((N_BATCH, 1, N_HW), jnp.int32),
            jax.ShapeDtypeStruct((1, 1), jnp.float32),
        ],
        scratch_shapes=[
            pltpu.VMEM((2, KT, N_HW), jnp.float32),
            pltpu.VMEM((K_CODES // 8, 8, 1), jnp.float32),
        ],
    )(x3, w)


@functools.cache
def _make_sc_gather():
    info = plsc.get_sparse_core_info()
    nw = info.num_cores * info.num_subcores
    b_per_w = (N_BATCH * N_HW) // nw
    mesh = plsc.VectorSubcoreMesh(core_axis_name="c", subcore_axis_name="s")

    @functools.partial(
        pl.kernel, mesh=mesh,
        out_type=jax.ShapeDtypeStruct((N_BATCH * N_HW, D_CH), jnp.float32),
        scratch_types=[
            pltpu.VMEM((b_per_w,), jnp.int32),
            pltpu.VMEM((b_per_w, D_CH), jnp.float32),
            pltpu.SemaphoreType.DMA,
        ],
    )
    def sc_gather(table_hbm, idx_hbm, out_hbm, idx_v, rows_v, sem):
        wid = lax.axis_index("s") * info.num_cores + lax.axis_index("c")
        base = wid * b_per_w
        pltpu.sync_copy(idx_hbm.at[pl.ds(base, b_per_w)], idx_v)
        pltpu.async_copy(table_hbm.at[idx_v], rows_v, sem).wait()
        pltpu.sync_copy(rows_v, out_hbm.at[pl.ds(base, b_per_w)])

    return sc_gather


def _transpose_body(q_ref, x_ref, o_ref):
    qt = q_ref[0].T
    xb = x_ref[0]
    o_ref[0] = xb + (qt - xb)


def _transpose_back(q, x3):
    return pl.pallas_call(
        _transpose_body,
        grid=(N_BATCH,),
        in_specs=[
            pl.BlockSpec((1, N_HW, D_CH), lambda b: (b, 0, 0)),
            pl.BlockSpec((1, D_CH, N_HW), lambda b: (b, 0, 0)),
        ],
        out_specs=pl.BlockSpec((1, D_CH, N_HW), lambda b: (b, 0, 0)),
        out_shape=jax.ShapeDtypeStruct((N_BATCH, D_CH, N_HW), jnp.float32),
    )(q, x3)


def kernel(x, embedding_weight):
    x3 = x.reshape(N_BATCH, D_CH, N_HW)
    idx3, loss_sum = _distance_argmin(x3, embedding_weight)
    idx_flat = idx3.reshape(N_BATCH * N_HW)
    q = _make_sc_gather()(embedding_weight, idx_flat)
    out = _transpose_back(q.reshape(N_BATCH, N_HW, D_CH), x3)
    out = out.reshape(N_BATCH, D_CH, 32, 32)
    loss = loss_sum[0, 0] * ((1.0 + COMMIT) / (N_BATCH * N_HW * D_CH))
    return (out, loss, idx_flat[:, None])


# speedup vs baseline: 2.1446x; 1.0383x over previous
import functools

import jax
import jax.numpy as jnp
from jax import lax
from jax.experimental import pallas as pl
from jax.experimental.pallas import tpu as pltpu
from jax.experimental.pallas import tpu_sc as plsc

K_CODES = 8192
D_CH = 256
N_BATCH = 8
N_HW = 1024
KT = 512
COMMIT = 0.25


B_STEP = 2


def _argmin_body(x_ref, w_ref, idx_ref, loss_ref, s2_ref, en8_ref):
    g = pl.program_id(0)

    @pl.when(g == 0)
    def _():
        loss_ref[0, 0] = 0.0
        for kt in range(K_CODES // KT):
            wt = w_ref[kt * KT:(kt + 1) * KT, :]
            en = jnp.sum(wt * wt, axis=1, keepdims=True)
            en8_ref[kt * (KT // 8):(kt + 1) * (KT // 8)] = en.reshape(KT // 8, 8, 1)

    vc_per_kt = KT // 8
    loss_part = jnp.zeros((1, N_HW), jnp.float32)
    for bb in range(B_STEP):
        xb = x_ref[bb]
        xn = jnp.sum(xb * xb, axis=0, keepdims=True)
        xm2 = -2.0 * xb

        m8 = jnp.full((8, N_HW), jnp.float32(jnp.inf))
        r8 = jnp.zeros((8, N_HW), jnp.int32)

        for kt in range(K_CODES // KT):
            wt = w_ref[kt * KT:(kt + 1) * KT, :]
            buf = kt % 2
            s2_ref[buf] = jnp.dot(wt, xm2, preferred_element_type=jnp.float32)

            for v in range(vc_per_kt):
                s2v = s2_ref[buf, v * 8:(v + 1) * 8, :]
                env = en8_ref[kt * vc_per_kt + v]
                d = (xn + env) + s2v
                take = d < m8
                m8 = jnp.where(take, d, m8)
                r8 = jnp.where(take, jnp.int32(kt * vc_per_kt + v), r8)

        tmin = jnp.min(m8, axis=0, keepdims=True)
        siota = lax.broadcasted_iota(jnp.int32, (8, N_HW), 0)
        gk = r8 * 8 + siota
        cand = jnp.where(m8 == tmin, gk, jnp.int32(2 ** 30))
        gidx = jnp.min(cand, axis=0, keepdims=True)
        idx_ref[bb] = gidx
        loss_part = loss_part + tmin
    loss_ref[0, 0] += jnp.sum(loss_part)


def _distance_argmin(x3, w):
    return pl.pallas_call(
        _argmin_body,
        grid=(N_BATCH // B_STEP,),
        in_specs=[
            pl.BlockSpec((B_STEP, D_CH, N_HW), lambda b: (b, 0, 0)),
            pl.BlockSpec((K_CODES, D_CH), lambda b: (0, 0)),
        ],
        out_specs=[
            pl.BlockSpec((B_STEP, 1, N_HW), lambda b: (b, 0, 0)),
            pl.BlockSpec(memory_space=pltpu.SMEM),
        ],
        out_shape=[
            jax.ShapeDtypeStruct((N_BATCH, 1, N_HW), jnp.int32),
            jax.ShapeDtypeStruct((1, 1), jnp.float32),
        ],
        scratch_shapes=[
            pltpu.VMEM((2, KT, N_HW), jnp.float32),
            pltpu.VMEM((K_CODES // 8, 8, 1), jnp.float32),
        ],
    )(x3, w)


@functools.cache
def _make_sc_gather():
    info = plsc.get_sparse_core_info()
    nw = info.num_cores * info.num_subcores
    b_per_w = (N_BATCH * N_HW) // nw
    mesh = plsc.VectorSubcoreMesh(core_axis_name="c", subcore_axis_name="s")

    @functools.partial(
        pl.kernel, mesh=mesh,
        out_type=jax.ShapeDtypeStruct((N_BATCH * N_HW, D_CH), jnp.float32),
        scratch_types=[
            pltpu.VMEM((b_per_w,), jnp.int32),
            pltpu.VMEM((b_per_w, D_CH), jnp.float32),
            pltpu.SemaphoreType.DMA,
        ],
    )
    def sc_gather(table_hbm, idx_hbm, out_hbm, idx_v, rows_v, sem):
        wid = lax.axis_index("s") * info.num_cores + lax.axis_index("c")
        base = wid * b_per_w
        pltpu.sync_copy(idx_hbm.at[pl.ds(base, b_per_w)], idx_v)
        pltpu.async_copy(table_hbm.at[idx_v], rows_v, sem).wait()
        pltpu.sync_copy(rows_v, out_hbm.at[pl.ds(base, b_per_w)])

    return sc_gather


def _transpose_body(q_ref, x_ref, o_ref):
    qt = q_ref[0].T
    xb = x_ref[0]
    o_ref[0] = xb + (qt - xb)


def _transpose_back(q, x3):
    return pl.pallas_call(
        _transpose_body,
        grid=(N_BATCH,),
        in_specs=[
            pl.BlockSpec((1, N_HW, D_CH), lambda b: (b, 0, 0)),
            pl.BlockSpec((1, D_CH, N_HW), lambda b: (b, 0, 0)),
        ],
        out_specs=pl.BlockSpec((1, D_CH, N_HW), lambda b: (b, 0, 0)),
        out_shape=jax.ShapeDtypeStruct((N_BATCH, D_CH, N_HW), jnp.float32),
    )(q, x3)


def kernel(x, embedding_weight):
    x3 = x.reshape(N_BATCH, D_CH, N_HW)
    idx3, loss_sum = _distance_argmin(x3, embedding_weight)
    idx_flat = idx3.reshape(N_BATCH * N_HW)
    q = _make_sc_gather()(embedding_weight, idx_flat)
    out = _transpose_back(q.reshape(N_BATCH, N_HW, D_CH), x3)
    out = out.reshape(N_BATCH, D_CH, 32, 32)
    loss = loss_sum[0, 0] * ((1.0 + COMMIT) / (N_BATCH * N_HW * D_CH))
    return (out, loss, idx_flat[:, None])
